# initial kernel scaffold (unmeasured)
import functools

import jax
import jax.numpy as jnp
from jax import lax
from jax.experimental import pallas as pl
from jax.experimental.pallas import tpu as pltpu

M = 4096
D = 4096
F_SHARD = 8192
M_LOCAL = M // 2
M_Q = M // 4

BN = 2048
BK = 1024


def _mm_body(y_ref, dy_ref, w_ref, out_ref):
    k = pl.program_id(1)

    @pl.when(k == 0)
    def _():
        out_ref[...] = jnp.zeros_like(out_ref)

    a = dy_ref[...].astype(jnp.bfloat16)
    b = w_ref[...].astype(jnp.bfloat16)
    out_ref[...] += lax.dot_general(
        a, b, (((1,), (1,)), ((), ())), preferred_element_type=jnp.float32
    )


def _partial_matmul(dy, w, my_y):
    grid_spec = pltpu.PrefetchScalarGridSpec(
        num_scalar_prefetch=1,
        grid=(D // BN, F_SHARD // BK),
        in_specs=[
            pl.BlockSpec((M_LOCAL, BK), lambda j, k, y: (y[0], k)),
            pl.BlockSpec((BN, BK), lambda j, k, y: (j, k)),
        ],
        out_specs=pl.BlockSpec((M_LOCAL, BN), lambda j, k, y: (0, j)),
    )
    return pl.pallas_call(
        _mm_body,
        grid_spec=grid_spec,
        out_shape=jax.ShapeDtypeStruct((M_LOCAL, D), jnp.float32),
        compiler_params=pltpu.CompilerParams(
            vmem_limit_bytes=100 * 1024 * 1024,
        ),
    )(my_y, dy, w)


def _comm_body(partial_ref, out_ref, cm, rs_buf, outbuf,
               send_sems, recv_sems, copy_sem):
    my_x = lax.axis_index("x")
    my_y = lax.axis_index("y")
    q_me = 2 * my_y + my_x
    q_px = 2 * my_y + (1 - my_x)

    barrier_sem = pltpu.get_barrier_semaphore()
    pl.semaphore_signal(barrier_sem, inc=1, device_id=(1 - my_x, my_y),
                        device_id_type=pl.DeviceIdType.MESH)
    pl.semaphore_signal(barrier_sem, inc=1, device_id=(my_x, 1 - my_y),
                        device_id_type=pl.DeviceIdType.MESH)
    pl.semaphore_wait(barrier_sem, 2)

    cm[q_px] = partial_ref[pl.ds((1 - my_x) * M_Q, M_Q), :].astype(jnp.bfloat16)
    rs = pltpu.make_async_remote_copy(
        src_ref=cm.at[q_px],
        dst_ref=rs_buf,
        send_sem=send_sems.at[0],
        recv_sem=recv_sems.at[0],
        device_id=(1 - my_x, my_y),
        device_id_type=pl.DeviceIdType.MESH,
    )
    rs.start()
    rs.wait()
    mine = partial_ref[pl.ds(my_x * M_Q, M_Q), :]
    cm[q_me] = (mine + rs_buf[...].astype(jnp.float32)).astype(jnp.bfloat16)

    ag1 = pltpu.make_async_remote_copy(
        src_ref=cm.at[q_me],
        dst_ref=cm.at[q_me],
        send_sem=send_sems.at[1],
        recv_sem=recv_sems.at[1],
        device_id=(1 - my_x, my_y),
        device_id_type=pl.DeviceIdType.MESH,
    )
    ag1.start()
    ag1.wait()

    ag2 = pltpu.make_async_remote_copy(
        src_ref=cm.at[pl.ds(2 * my_y, 2)],
        dst_ref=cm.at[pl.ds(2 * my_y, 2)],
        send_sem=send_sems.at[2],
        recv_sem=recv_sems.at[2],
        device_id=(my_x, 1 - my_y),
        device_id_type=pl.DeviceIdType.MESH,
    )
    ag2.start()
    ag2.wait()

    for q in range(4):
        outbuf[...] = cm[q].astype(jnp.float32)
        cp = pltpu.make_async_copy(
            outbuf, out_ref.at[pl.ds(q * M_Q, M_Q), :], copy_sem
        )
        cp.start()
        cp.wait()


def _allreduce_gather(partial):
    return pl.pallas_call(
        _comm_body,
        out_shape=jax.ShapeDtypeStruct((M, D), jnp.float32),
        in_specs=[pl.BlockSpec(memory_space=pltpu.VMEM)],
        out_specs=pl.BlockSpec(memory_space=pltpu.ANY),
        scratch_shapes=[
            pltpu.VMEM((4, M_Q, D), jnp.bfloat16),
            pltpu.VMEM((M_Q, D), jnp.bfloat16),
            pltpu.VMEM((M_Q, D), jnp.float32),
            pltpu.SemaphoreType.DMA((3,)),
            pltpu.SemaphoreType.DMA((3,)),
            pltpu.SemaphoreType.DMA,
        ],
        compiler_params=pltpu.CompilerParams(
            collective_id=0,
            vmem_limit_bytes=110 * 1024 * 1024,
        ),
    )(partial)


def kernel(dy, W):
    my_y = lax.axis_index("y")
    y_scalar = jnp.reshape(my_y, (1,)).astype(jnp.int32)
    partial = _partial_matmul(dy, W, y_scalar)
    return _allreduce_gather(partial)


# baseline (device time: 610912 ns/iter reference)
import jax
import jax.numpy as jnp
from jax import lax
from jax.experimental import pallas as pl
from jax.experimental.pallas import tpu as pltpu

M = 4096
D = 4096
F_SHARD = 8192
M_LOCAL = M // 2
M_Q = M // 4

BN = 2048
BK = 512
N_K = F_SHARD // BK


def _mm_body(y_ref, dy_ref, w_ref, out_ref, acc_ref):
    k = pl.program_id(1)

    @pl.when(k == 0)
    def _():
        acc_ref[...] = jnp.zeros_like(acc_ref)

    a = dy_ref[...].astype(jnp.bfloat16)
    b = w_ref[...].astype(jnp.bfloat16)
    acc_ref[...] += lax.dot_general(
        a, b, (((1,), (1,)), ((), ())), preferred_element_type=jnp.float32
    )

    @pl.when(k == N_K - 1)
    def _():
        out_ref[...] = acc_ref[...].astype(jnp.bfloat16)


def _partial_matmul(dy, w, my_y):
    grid_spec = pltpu.PrefetchScalarGridSpec(
        num_scalar_prefetch=1,
        grid=(D // BN, N_K),
        in_specs=[
            pl.BlockSpec((M_LOCAL, BK), lambda j, k, y: (y[0], k)),
            pl.BlockSpec((BN, BK), lambda j, k, y: (j, k)),
        ],
        out_specs=pl.BlockSpec((M_LOCAL, BN), lambda j, k, y: (0, j)),
        scratch_shapes=[pltpu.VMEM((M_LOCAL, BN), jnp.float32)],
    )
    return pl.pallas_call(
        _mm_body,
        grid_spec=grid_spec,
        out_shape=jax.ShapeDtypeStruct((M_LOCAL, D), jnp.bfloat16),
        compiler_params=pltpu.CompilerParams(
            vmem_limit_bytes=60 * 1024 * 1024,
        ),
    )(my_y, dy, w)


def _comm_body(partial_ref, out_ref, cm, rs_buf, outbuf,
               send_sems, recv_sems, copy_sems):
    my_x = lax.axis_index("x")
    my_y = lax.axis_index("y")
    q_me = 2 * my_y + my_x
    q_px = 2 * my_y + (1 - my_x)

    cp_peer = pltpu.make_async_copy(
        partial_ref.at[pl.ds((1 - my_x) * M_Q, M_Q), :],
        cm.at[q_px], copy_sems.at[0],
    )
    cp_mine = pltpu.make_async_copy(
        partial_ref.at[pl.ds(my_x * M_Q, M_Q), :],
        cm.at[q_me], copy_sems.at[1],
    )
    cp_peer.start()
    cp_mine.start()

    barrier_sem = pltpu.get_barrier_semaphore()
    pl.semaphore_signal(barrier_sem, inc=1, device_id=(1 - my_x, my_y),
                        device_id_type=pl.DeviceIdType.MESH)
    pl.semaphore_signal(barrier_sem, inc=1, device_id=(my_x, 1 - my_y),
                        device_id_type=pl.DeviceIdType.MESH)
    pl.semaphore_wait(barrier_sem, 2)

    cp_peer.wait()
    rs = pltpu.make_async_remote_copy(
        src_ref=cm.at[q_px],
        dst_ref=rs_buf,
        send_sem=send_sems.at[0],
        recv_sem=recv_sems.at[0],
        device_id=(1 - my_x, my_y),
        device_id_type=pl.DeviceIdType.MESH,
    )
    rs.start()
    cp_mine.wait()
    rs.wait()
    cm[q_me] = (cm[q_me].astype(jnp.float32)
                + rs_buf[...].astype(jnp.float32)).astype(jnp.bfloat16)

    ag1 = pltpu.make_async_remote_copy(
        src_ref=cm.at[q_me],
        dst_ref=cm.at[q_me],
        send_sem=send_sems.at[1],
        recv_sem=recv_sems.at[1],
        device_id=(1 - my_x, my_y),
        device_id_type=pl.DeviceIdType.MESH,
    )
    ag1.start()
    ag1.wait()

    ag2 = pltpu.make_async_remote_copy(
        src_ref=cm.at[pl.ds(2 * my_y, 2)],
        dst_ref=cm.at[pl.ds(2 * my_y, 2)],
        send_sem=send_sems.at[2],
        recv_sem=recv_sems.at[2],
        device_id=(my_x, 1 - my_y),
        device_id_type=pl.DeviceIdType.MESH,
    )
    ag2.start()
    ag2.wait()

    for q in range(4):
        outbuf[...] = cm[q].astype(jnp.float32)
        cp = pltpu.make_async_copy(
            outbuf, out_ref.at[pl.ds(q * M_Q, M_Q), :], copy_sems.at[0]
        )
        cp.start()
        cp.wait()


def _allreduce_gather(partial):
    return pl.pallas_call(
        _comm_body,
        out_shape=jax.ShapeDtypeStruct((M, D), jnp.float32),
        in_specs=[pl.BlockSpec(memory_space=pl.ANY)],
        out_specs=pl.BlockSpec(memory_space=pl.ANY),
        scratch_shapes=[
            pltpu.VMEM((4, M_Q, D), jnp.bfloat16),
            pltpu.VMEM((M_Q, D), jnp.bfloat16),
            pltpu.VMEM((M_Q, D), jnp.float32),
            pltpu.SemaphoreType.DMA((3,)),
            pltpu.SemaphoreType.DMA((3,)),
            pltpu.SemaphoreType.DMA((2,)),
        ],
        compiler_params=pltpu.CompilerParams(
            collective_id=0,
            vmem_limit_bytes=60 * 1024 * 1024,
        ),
    )(partial)


def kernel(dy, W):
    my_y = lax.axis_index("y")
    y_scalar = jnp.reshape(my_y, (1,)).astype(jnp.int32)
    partial = _partial_matmul(dy, W, y_scalar)
    return _allreduce_gather(partial)


# device time: 460720 ns/iter; 1.3260x vs baseline; 1.3260x over previous
import jax
import jax.numpy as jnp
from jax import lax
from jax.experimental import pallas as pl
from jax.experimental.pallas import tpu as pltpu

M = 4096
D = 4096
F_SHARD = 8192
M_LOCAL = M // 2
M_Q = M // 4

BN = 2048
BK = 512
N_K = F_SHARD // BK


def _mm_body(y_ref, dy_ref, w_ref, out_ref, acc_ref):
    k = pl.program_id(1)

    @pl.when(k == 0)
    def _():
        acc_ref[...] = jnp.zeros_like(acc_ref)

    a = dy_ref[...].astype(jnp.bfloat16)
    b = w_ref[...].astype(jnp.bfloat16)
    acc_ref[...] += lax.dot_general(
        a, b, (((1,), (1,)), ((), ())), preferred_element_type=jnp.float32
    )

    @pl.when(k == N_K - 1)
    def _():
        out_ref[...] = acc_ref[...].astype(jnp.bfloat16)


def _partial_matmul(dy, w, my_y):
    grid_spec = pltpu.PrefetchScalarGridSpec(
        num_scalar_prefetch=1,
        grid=(D // BN, N_K),
        in_specs=[
            pl.BlockSpec((M_LOCAL, BK), lambda j, k, y: (y[0], k)),
            pl.BlockSpec((BN, BK), lambda j, k, y: (j, k)),
        ],
        out_specs=pl.BlockSpec((M_LOCAL, BN), lambda j, k, y: (0, j)),
        scratch_shapes=[pltpu.VMEM((M_LOCAL, BN), jnp.float32)],
    )
    return pl.pallas_call(
        _mm_body,
        grid_spec=grid_spec,
        out_shape=jax.ShapeDtypeStruct((M_LOCAL, D), jnp.bfloat16),
        compiler_params=pltpu.CompilerParams(
            vmem_limit_bytes=60 * 1024 * 1024,
        ),
    )(my_y, dy, w)


def _comm_body(partial_ref, out_ref, cm, rs_buf, outbuf,
               send_sems, recv_sems, copy_sems):
    my_x = lax.axis_index("x")
    my_y = lax.axis_index("y")
    q_me = 2 * my_y + my_x
    q_px = 2 * my_y + (1 - my_x)

    cp_peer = pltpu.make_async_copy(
        partial_ref.at[pl.ds((1 - my_x) * M_Q, M_Q), :],
        cm.at[q_px], copy_sems.at[0],
    )
    cp_mine = pltpu.make_async_copy(
        partial_ref.at[pl.ds(my_x * M_Q, M_Q), :],
        cm.at[q_me], copy_sems.at[1],
    )
    cp_peer.start()
    cp_mine.start()

    barrier_sem = pltpu.get_barrier_semaphore()
    pl.semaphore_signal(barrier_sem, inc=1, device_id=(1 - my_x, my_y),
                        device_id_type=pl.DeviceIdType.MESH)
    pl.semaphore_signal(barrier_sem, inc=1, device_id=(my_x, 1 - my_y),
                        device_id_type=pl.DeviceIdType.MESH)
    pl.semaphore_wait(barrier_sem, 2)

    cp_peer.wait()
    rs = pltpu.make_async_remote_copy(
        src_ref=cm.at[q_px],
        dst_ref=rs_buf,
        send_sem=send_sems.at[0],
        recv_sem=recv_sems.at[0],
        device_id=(1 - my_x, my_y),
        device_id_type=pl.DeviceIdType.MESH,
    )
    rs.start()
    cp_mine.wait()
    rs.wait()
    cm[q_me] = (cm[q_me].astype(jnp.float32)
                + rs_buf[...].astype(jnp.float32)).astype(jnp.bfloat16)

    ag1 = pltpu.make_async_remote_copy(
        src_ref=cm.at[q_me],
        dst_ref=cm.at[q_me],
        send_sem=send_sems.at[1],
        recv_sem=recv_sems.at[1],
        device_id=(1 - my_x, my_y),
        device_id_type=pl.DeviceIdType.MESH,
    )
    ag1.start()
    ag1.wait()

    ag2 = pltpu.make_async_remote_copy(
        src_ref=cm.at[pl.ds(2 * my_y, 2)],
        dst_ref=cm.at[pl.ds(2 * my_y, 2)],
        send_sem=send_sems.at[2],
        recv_sem=recv_sems.at[2],
        device_id=(my_x, 1 - my_y),
        device_id_type=pl.DeviceIdType.MESH,
    )
    ag2.start()
    ag2.wait()

    for q in range(4):
        outbuf[...] = cm[q].astype(jnp.float32)
        cp = pltpu.make_async_copy(
            outbuf, out_ref.at[pl.ds(q * M_Q, M_Q), :], copy_sems.at[0]
        )
        cp.start()
        cp.wait()


def _allreduce_gather(partial):
    return pl.pallas_call(
        _comm_body,
        out_shape=jax.ShapeDtypeStruct((M, D), jnp.float32),
        in_specs=[pl.BlockSpec(memory_space=pl.ANY)],
        out_specs=pl.BlockSpec(memory_space=pl.ANY),
        scratch_shapes=[
            pltpu.VMEM((4, M_Q, D), jnp.bfloat16),
            pltpu.VMEM((M_Q, D), jnp.bfloat16),
            pltpu.VMEM((M_Q, D), jnp.float32),
            pltpu.SemaphoreType.DMA((3,)),
            pltpu.SemaphoreType.DMA((3,)),
            pltpu.SemaphoreType.DMA((2,)),
        ],
        compiler_params=pltpu.CompilerParams(
            collective_id=0,
            vmem_limit_bytes=60 * 1024 * 1024,
        ),
    )(partial)


def kernel(dy, W):
    import os
    part = os.environ.get("KERNEL_PART", "")
    if part == "mm":
        my_y = lax.axis_index("y")
        y_scalar = jnp.reshape(my_y, (1,)).astype(jnp.int32)
        partial = _partial_matmul(dy, W, y_scalar)
        return jnp.concatenate([partial, partial], 0).astype(jnp.float32)
    if part == "comm":
        partial = dy[:M_LOCAL, :D].astype(jnp.bfloat16)
        return _allreduce_gather(partial)
    my_y = lax.axis_index("y")
    y_scalar = jnp.reshape(my_y, (1,)).astype(jnp.int32)
    partial = _partial_matmul(dy, W, y_scalar)
    return _allreduce_gather(partial)


# device time: 278389 ns/iter; 2.1945x vs baseline; 1.6550x over previous
import jax
import jax.numpy as jnp
from jax import lax
from jax.experimental import pallas as pl
from jax.experimental.pallas import tpu as pltpu

M = 4096
D = 4096
F_SHARD = 8192
M_LOCAL = M // 2

BN = 2048
BK = 512
N_K = F_SHARD // BK

S = 8
ROWS = M_LOCAL // S


def _mm_body(y_ref, dy_ref, w_ref, out_ref, acc_ref):
    k = pl.program_id(1)

    @pl.when(k == 0)
    def _():
        acc_ref[...] = jnp.zeros_like(acc_ref)

    a = dy_ref[...].astype(jnp.bfloat16)
    b = w_ref[...].astype(jnp.bfloat16)
    acc_ref[...] += lax.dot_general(
        a, b, (((1,), (1,)), ((), ())), preferred_element_type=jnp.float32
    )

    @pl.when(k == N_K - 1)
    def _():
        out_ref[...] = acc_ref[...].astype(jnp.bfloat16)


def _partial_matmul(dy, w, my_y):
    grid_spec = pltpu.PrefetchScalarGridSpec(
        num_scalar_prefetch=1,
        grid=(D // BN, N_K),
        in_specs=[
            pl.BlockSpec((M_LOCAL, BK), lambda j, k, y: (y[0], k)),
            pl.BlockSpec((BN, BK), lambda j, k, y: (j, k)),
        ],
        out_specs=pl.BlockSpec((M_LOCAL, BN), lambda j, k, y: (0, j)),
        scratch_shapes=[pltpu.VMEM((M_LOCAL, BN), jnp.float32)],
    )
    return pl.pallas_call(
        _mm_body,
        grid_spec=grid_spec,
        out_shape=jax.ShapeDtypeStruct((M_LOCAL, D), jnp.bfloat16),
        compiler_params=pltpu.CompilerParams(
            vmem_limit_bytes=60 * 1024 * 1024,
        ),
    )(my_y, dy, w)


def _comm_body(partial_ref, out_ref, pbuf, xrecv, yrecv, outstage,
               x_send_sems, x_recv_sems, y_send_sems, y_recv_sems,
               in_sem, out_sems):
    my_x = lax.axis_index("x")
    my_y = lax.axis_index("y")

    cp_in = pltpu.make_async_copy(partial_ref, pbuf, in_sem)
    cp_in.start()

    barrier_sem = pltpu.get_barrier_semaphore()
    pl.semaphore_signal(barrier_sem, inc=1, device_id=(1 - my_x, my_y),
                        device_id_type=pl.DeviceIdType.MESH)
    pl.semaphore_signal(barrier_sem, inc=1, device_id=(my_x, 1 - my_y),
                        device_id_type=pl.DeviceIdType.MESH)
    pl.semaphore_wait(barrier_sem, 2)
    cp_in.wait()

    x_rdmas = []
    for s in range(S):
        sl = pl.ds(s * ROWS, ROWS)
        r = pltpu.make_async_remote_copy(
            src_ref=pbuf.at[sl],
            dst_ref=xrecv.at[sl],
            send_sem=x_send_sems.at[s],
            recv_sem=x_recv_sems.at[s],
            device_id=(1 - my_x, my_y),
            device_id_type=pl.DeviceIdType.MESH,
        )
        r.start()
        x_rdmas.append(r)

    last_out = [None, None]

    def _stage_out(val_bf16, row_start, slot):
        if last_out[slot] is not None:
            last_out[slot].wait()
        outstage[slot] = val_bf16.astype(jnp.float32)
        cp = pltpu.make_async_copy(
            outstage.at[slot], out_ref.at[pl.ds(row_start, ROWS), :],
            out_sems.at[slot],
        )
        cp.start()
        last_out[slot] = cp

    y_rdmas = []
    for s in range(S):
        sl = pl.ds(s * ROWS, ROWS)
        x_rdmas[s].wait()
        red = (pbuf[sl, :].astype(jnp.float32)
               + xrecv[sl, :].astype(jnp.float32)).astype(jnp.bfloat16)
        pbuf[sl, :] = red
        r = pltpu.make_async_remote_copy(
            src_ref=pbuf.at[sl],
            dst_ref=yrecv.at[sl],
            send_sem=y_send_sems.at[s],
            recv_sem=y_recv_sems.at[s],
            device_id=(my_x, 1 - my_y),
            device_id_type=pl.DeviceIdType.MESH,
        )
        r.start()
        y_rdmas.append(r)
        _stage_out(red, my_y * M_LOCAL + s * ROWS, s % 2)

    for s in range(S):
        sl = pl.ds(s * ROWS, ROWS)
        y_rdmas[s].wait()
        _stage_out(yrecv[sl, :], (1 - my_y) * M_LOCAL + s * ROWS, s % 2)

    last_out[0].wait()
    last_out[1].wait()


def _allreduce_gather(partial):
    return pl.pallas_call(
        _comm_body,
        out_shape=jax.ShapeDtypeStruct((M, D), jnp.float32),
        in_specs=[pl.BlockSpec(memory_space=pl.ANY)],
        out_specs=pl.BlockSpec(memory_space=pl.ANY),
        scratch_shapes=[
            pltpu.VMEM((M_LOCAL, D), jnp.bfloat16),
            pltpu.VMEM((M_LOCAL, D), jnp.bfloat16),
            pltpu.VMEM((M_LOCAL, D), jnp.bfloat16),
            pltpu.VMEM((2, ROWS, D), jnp.float32),
            pltpu.SemaphoreType.DMA((S,)),
            pltpu.SemaphoreType.DMA((S,)),
            pltpu.SemaphoreType.DMA((S,)),
            pltpu.SemaphoreType.DMA((S,)),
            pltpu.SemaphoreType.DMA,
            pltpu.SemaphoreType.DMA((2,)),
        ],
        compiler_params=pltpu.CompilerParams(
            collective_id=0,
            vmem_limit_bytes=60 * 1024 * 1024,
        ),
    )(partial)


def kernel(dy, W):
    import os
    part = os.environ.get("KERNEL_PART", "")
    my_y = lax.axis_index("y")
    y_scalar = jnp.reshape(my_y, (1,)).astype(jnp.int32)
    if part == "mm":
        partial = _partial_matmul(dy, W, y_scalar)
        return jnp.concatenate([partial, partial], 0).astype(jnp.float32)
    if part == "comm":
        partial = dy[:M_LOCAL, :D].astype(jnp.bfloat16)
        return _allreduce_gather(partial)
    partial = _partial_matmul(dy, W, y_scalar)
    return _allreduce_gather(partial)
